# baseline (speedup 1.0000x reference)
import jax
import jax.numpy as jnp
from jax import lax
from jax.experimental import pallas as pl
from jax.experimental.pallas import tpu as pltpu

_C = 1024
_D = 4
_MAXN = 64


def _body(bs_ref, cs_ref, vs_ref, nn_ref, yp_ref, yt_ref, out_ref,
          bp, bt, accr, semp, semt):
    i32 = jnp.int32
    n = nn_ref[0]

    def _fire(idx, slot):
        idx = jnp.minimum(idx, i32(_MAXN - 1))
        b = bs_ref[idx]
        c = cs_ref[idx]
        src_p = yp_ref.at[b, :, :, pl.ds(c * _C, _C)]
        pltpu.make_async_copy(src_p, bp.at[slot], semp.at[slot]).start()
        src_t = yt_ref.at[b, :, :, pl.ds(c * _C, _C)]
        pltpu.make_async_copy(src_t, bt.at[slot], semt.at[slot]).start()

    def _drain(slot):
        dummy = yp_ref.at[0, :, :, pl.ds(0, _C)]
        pltpu.make_async_copy(dummy, bp.at[slot], semp.at[slot]).wait()
        pltpu.make_async_copy(dummy, bt.at[slot], semt.at[slot]).wait()

    accr[...] = jnp.zeros_like(accr)
    for slot in range(_D):
        _fire(i32(slot), slot)

    nouter = (n + (_D - 1)) >> 2
    lane = lax.broadcasted_iota(i32, (4, 4, _C), 2)

    def outer(it, _):
        base = it * _D
        for slot in range(_D):
            idx = base + slot
            _drain(slot)
            v = vs_ref[jnp.minimum(idx, i32(_MAXN - 1))]
            msk = lane < v
            d = bp[slot] - bt[slot]
            accr[...] += jnp.where(msk, d * d, 0.0)

        @pl.when(it + 1 < nouter)
        def _():
            for slot in range(_D):
                _fire(base + _D + slot, slot)
        return 0

    lax.fori_loop(0, nouter, outer, 0)
    out_ref[0, 0] = jnp.sum(accr[...])


def kernel(y_pred, y_true, lengths):
    yp = jnp.transpose(y_pred, (0, 2, 3, 1))
    yt = jnp.transpose(y_true, (0, 2, 3, 1))
    thr = jnp.maximum(lengths.astype(jnp.int32) - 1, 0)

    nb = (thr + (_C - 1)) // _C
    cum = jnp.cumsum(nb)
    n = cum[-1]
    g = jnp.arange(_MAXN, dtype=jnp.int32)
    gc = jnp.minimum(g, jnp.maximum(n - 1, 0))
    bs = jnp.minimum(jnp.searchsorted(cum, gc, side="right").astype(jnp.int32), 15)
    cs = gc - (cum[bs] - nb[bs])
    vs = jnp.where(g < n, jnp.minimum(thr[bs] - cs * _C, _C), 0).astype(jnp.int32)
    nn = jnp.full((1,), n, jnp.int32)

    grid_spec = pltpu.PrefetchScalarGridSpec(
        num_scalar_prefetch=4,
        grid=(1,),
        in_specs=[
            pl.BlockSpec(memory_space=pl.ANY),
            pl.BlockSpec(memory_space=pl.ANY),
        ],
        out_specs=pl.BlockSpec(memory_space=pltpu.SMEM),
        scratch_shapes=[
            pltpu.VMEM((_D, 4, 4, _C), jnp.float32),
            pltpu.VMEM((_D, 4, 4, _C), jnp.float32),
            pltpu.VMEM((4, 4, _C), jnp.float32),
            pltpu.SemaphoreType.DMA((_D,)),
            pltpu.SemaphoreType.DMA((_D,)),
        ],
    )
    out = pl.pallas_call(
        _body,
        grid_spec=grid_spec,
        out_shape=jax.ShapeDtypeStruct((1, 1), jnp.float32),
    )(bs, cs, vs, nn, yp, yt)

    cnt = (jnp.sum(thr) * 16).astype(jnp.float32)
    return out[0, 0] / cnt


# speedup vs baseline: 3.2477x; 3.2477x over previous
import jax
import jax.numpy as jnp
from jax import lax
from jax.experimental import pallas as pl
from jax.experimental.pallas import tpu as pltpu

_Q = 4
_D = 2


def _body(thr_ref, yp_ref, yt_ref, out_ref, bp, bt, accr, semp, semt):
    i32 = jnp.int32
    n = yp_ref.shape[3]

    def _fire(c, slot):
        src_p = yp_ref.at[pl.ds(c * 4, 4)]
        pltpu.make_async_copy(src_p, bp.at[slot], semp.at[slot]).start()
        src_t = yt_ref.at[pl.ds(c * 4, 4)]
        pltpu.make_async_copy(src_t, bt.at[slot], semt.at[slot]).start()

    def _drain(slot):
        dummy = yp_ref.at[pl.ds(0, 4)]
        pltpu.make_async_copy(dummy, bp.at[slot], semp.at[slot]).wait()
        pltpu.make_async_copy(dummy, bt.at[slot], semt.at[slot]).wait()

    accr[...] = jnp.zeros_like(accr)
    lane = lax.broadcasted_iota(i32, (4, 4, n), 2)
    for slot in range(_D):
        _fire(slot, slot)
    for c in range(_Q):
        slot = c % _D
        _drain(slot)
        for rr in range(4):
            thr_b = thr_ref[c * 4 + rr]
            msk = lane < thr_b
            d = bp[slot, rr] - bt[slot, rr]
            accr[...] += jnp.where(msk, d * d, 0.0)
        if c + _D < _Q:
            _fire(c + _D, slot)
    cnt = i32(0)
    for b in range(16):
        cnt = cnt + thr_ref[b]
    out_ref[0, 0] = jnp.sum(accr[...]) / (cnt.astype(jnp.float32) * 16.0)


def kernel(y_pred, y_true, lengths):
    yp = jnp.transpose(y_pred, (0, 2, 3, 1))
    yt = jnp.transpose(y_true, (0, 2, 3, 1))
    thr = jnp.maximum(lengths.astype(jnp.int32) - 1, 0)
    n = yp.shape[3]

    grid_spec = pltpu.PrefetchScalarGridSpec(
        num_scalar_prefetch=1,
        grid=(1,),
        in_specs=[
            pl.BlockSpec(memory_space=pl.ANY),
            pl.BlockSpec(memory_space=pl.ANY),
        ],
        out_specs=pl.BlockSpec(memory_space=pltpu.SMEM),
        scratch_shapes=[
            pltpu.VMEM((_D, 4, 4, 4, n), jnp.float32),
            pltpu.VMEM((_D, 4, 4, 4, n), jnp.float32),
            pltpu.VMEM((4, 4, n), jnp.float32),
            pltpu.SemaphoreType.DMA((_D,)),
            pltpu.SemaphoreType.DMA((_D,)),
        ],
    )
    out = pl.pallas_call(
        _body,
        grid_spec=grid_spec,
        out_shape=jax.ShapeDtypeStruct((1, 1), jnp.float32),
    )(thr, yp, yt)
    return out[0, 0]
